# baseline (speedup 1.0000x reference)
import functools

import jax
import jax.numpy as jnp
from jax import lax
from jax.experimental import pallas as pl
from jax.experimental.pallas import tpu as pltpu
from jax.experimental.pallas import tpu_sc as plsc

_FACTOR = 6.26
_TOPK = 128
_NB = 16
_UNROLL = 8


def _zero_hist(hist_ref, base, nwords):
    z = jnp.zeros((_NB,), jnp.int32)

    def body(i, c):
        off = base + i * (_NB * 8)
        for j in range(8):
            hist_ref[pl.ds(off + j * _NB, _NB)] = z
        return c

    lax.fori_loop(0, nwords // (_NB * 8), body, 0)


def _find_bucket(hist_ref, base, r, nbuckets, ascending):
    nch = nbuckets // _NB

    def chunk_sum(c):
        off = base + c * (_NB * _NB)
        acc = hist_ref[pl.ds(off, _NB)]
        for j in range(1, _NB):
            acc = acc + hist_ref[pl.ds(off + j * _NB, _NB)]
        return jnp.sum(acc)

    def chunk_body(i, carry):
        cum, cstar, cnt_hi = carry
        c = i if ascending else (nch - 1) - i
        tot = chunk_sum(c)
        ncum = cum + tot
        hit = (cstar < 0) & (ncum >= r)
        cstar = jnp.where(hit, c, cstar)
        cnt_hi = jnp.where(hit, cum, cnt_hi)
        return ncum, cstar, cnt_hi

    cum_all, cstar, cnt_hi = lax.fori_loop(
        0, nch, chunk_body, (jnp.int32(0), jnp.int32(-1), jnp.int32(0)))

    safe_cstar = jnp.maximum(cstar, 0)

    def bucket_body(i, carry):
        cum, rstar, cnt_hi2, cnt_at = carry
        row = safe_cstar * _NB + (i if ascending else (_NB - 1) - i)
        tot = jnp.sum(hist_ref[pl.ds(base + row * _NB, _NB)])
        ncum = cum + tot
        hit = (rstar < 0) & (ncum >= r)
        rstar = jnp.where(hit, row, rstar)
        cnt_hi2 = jnp.where(hit, cum, cnt_hi2)
        cnt_at = jnp.where(hit, tot, cnt_at)
        return ncum, rstar, cnt_hi2, cnt_at

    _, rstar, cnt_hi2, cnt_at = lax.fori_loop(
        0, _NB, bucket_body,
        (cnt_hi, jnp.int32(-1), jnp.int32(0), jnp.int32(0)))
    rstar = jnp.where(cstar < 0, jnp.int32(-1), rstar)

    row0 = (nbuckets - 1) if ascending else 0
    tot0 = jnp.sum(hist_ref[pl.ds(base + row0 * _NB, _NB)])
    found = rstar >= 0
    bucket = jnp.where(found,
                       ((nbuckets - 1) - rstar) if ascending else rstar,
                       jnp.int32(0))
    cnt_hi_out = jnp.where(found, cnt_hi2, cum_all - tot0)
    cnt_at = jnp.where(found, cnt_at, tot0)
    return bucket, cnt_hi_out, cnt_at


_LEVELS = ((15, 256, 7, 256, False), (7, 256, 0, 128, False),
           (0, 128, 0, 0, True))


def _refine(cand, hist, c1, r, tbits, d):
    lane = lax.iota(jnp.int32, _NB)
    ones = jnp.full((_NB,), 1, jnp.int32)
    zi = jnp.zeros((_NB,), jnp.int32)
    zf = jnp.zeros((_NB,), jnp.float32)

    _zero_hist(hist, 0, 4096)
    nfull = c1 // _NB
    rem = c1 - nfull * _NB

    def h2_body(i, c):
        bits = cand[pl.ds(i * _NB, _NB)]
        a = (((bits >> 15) & 255) << 4) | lane
        plsc.addupdate_scatter(hist, [a], ones, mask=None)
        return c

    lax.fori_loop(0, nfull, h2_body, 0)
    bits = cand[pl.ds(nfull * _NB, _NB)]
    a = (((bits >> 15) & 255) << 4) | lane
    plsc.addupdate_scatter(hist, [a], ones, mask=lane < rem)

    cnt_gt = jnp.int32(0)
    n_eq = jnp.int32(0)
    ws = jnp.float32(0.0)
    cnum = c1
    hbase, hnext = 0, 4096
    for shift, width, nshift, nwidth, last in _LEVELS:
        bl, cnt_hi, cnt_at = _find_bucket(hist, hbase, r, width, False)
        tbits = tbits | (bl << shift)
        cnt_gt = cnt_gt + cnt_hi
        r = r - cnt_hi
        if last:
            n_eq = cnt_at
        if not last:
            _zero_hist(hist, hnext, 4096)
        nfull = cnum // _NB
        rem = cnum - nfull * _NB

        def sel_chunk(bits, valid, carry, shift=shift, width=width,
                      nshift=nshift, nwidth=nwidth, bl=bl, last=last,
                      hnext=hnext):
            offv, wsv = carry
            bk = (bits >> shift) & (width - 1)
            wsv = wsv + jnp.where(valid & (bk > bl),
                                  lax.bitcast_convert_type(bits, jnp.float32),
                                  0.0)
            if not last:
                m = valid & (bk == bl)
                mi = m.astype(jnp.int32)
                tgt = offv + (plsc.cumsum(mi) - mi)
                plsc.store_scatter(cand, [tgt], bits, mask=m)
                a2 = ((((bits >> nshift) & (nwidth - 1)) << 4) | lane) + hnext
                plsc.addupdate_scatter(hist, [a2], ones, mask=m)
                offv = offv + plsc.all_reduce_population_count(m)
            return offv, wsv

        def sel_body(i, carry, sel_chunk=sel_chunk):
            bits = cand[pl.ds(i * _NB, _NB)]
            return sel_chunk(bits, jnp.full((_NB,), True), carry)

        offv, wsv = lax.fori_loop(0, nfull, sel_body, (zi, zf))
        tail = cand[pl.ds(nfull * _NB, _NB)]
        offv, wsv = sel_chunk(tail, lane < rem, (offv, wsv))
        ws = ws + jnp.sum(wsv)
        if not last:
            cnum = jnp.max(offv)
        hbase, hnext = hnext, hbase

    return tbits, cnt_gt, n_eq, ws


def _process_row(xv, cand_a, cand_b, hist1, hist2, k1, k2, factor, d):
    lane = lax.iota(jnp.int32, _NB)
    ones = jnp.full((_NB,), 1, jnp.int32)
    _zero_hist(hist1, 0, 8192)
    u = _UNROLL if d % (_NB * _UNROLL) == 0 else 1

    def pass_a(i, carry):
        sv, sa = carry
        bs = i * (_NB * u)
        for j in range(u):
            v = xv[pl.ds(bs + j * _NB, _NB)]
            braw = lax.bitcast_convert_type(v, jnp.int32)
            babs = braw & 0x7FFFFFFF
            av = lax.bitcast_convert_type(babs, jnp.float32)
            sv = sv + v
            sa = sa + av
            bucket = babs >> 23
            row = jnp.where(v >= 0.0, 256 + bucket, 255 - bucket)
            addr = (row << 4) | lane
            plsc.addupdate_scatter(hist1, [addr], ones, mask=None)
        return sv, sa

    zf = jnp.zeros((_NB,), jnp.float32)
    zi = jnp.zeros((_NB,), jnp.int32)
    sv, sa = lax.fori_loop(0, d // (_NB * u), pass_a, (zf, zf))
    tsv = jnp.sum(sv)
    tsa = jnp.sum(sa)
    sum_p = 0.5 * (tsa + tsv)
    sum_n = 0.5 * (tsa - tsv)

    b1p, hi1p, _ = _find_bucket(hist1, 4096, k1, 256, False)
    b1n, hi1n, _ = _find_bucket(hist1, 0, k2, 256, True)

    def collect(i, carry):
        offp, offn, wsp, wsn = carry
        bs = i * (_NB * u)
        for j in range(u):
            v = xv[pl.ds(bs + j * _NB, _NB)]
            braw = lax.bitcast_convert_type(v, jnp.int32)
            babs = braw & 0x7FFFFFFF
            pos = v > 0.0
            neg = v < 0.0
            bkp = jnp.where(pos, babs, 0)
            bkn = jnp.where(neg, babs, 0)
            bucketp = bkp >> 23
            bucketn = bkn >> 23
            mp = bucketp == b1p
            mn = bucketn == b1n
            mpi = mp.astype(jnp.int32)
            mni = mn.astype(jnp.int32)
            tgtp = offp + (plsc.cumsum(mpi) - mpi)
            tgtn = offn + (plsc.cumsum(mni) - mni)
            plsc.store_scatter(cand_a, [tgtp], bkp, mask=mp)
            plsc.store_scatter(cand_b, [tgtn], bkn, mask=mn)
            offp = offp + plsc.all_reduce_population_count(mp)
            offn = offn + plsc.all_reduce_population_count(mn)
            pv = lax.bitcast_convert_type(bkp, jnp.float32)
            nv = lax.bitcast_convert_type(bkn, jnp.float32)
            wsp = wsp + jnp.where(bucketp > b1p, pv, 0.0)
            wsn = wsn + jnp.where(bucketn > b1n, nv, 0.0)
        return offp, offn, wsp, wsn

    offp, offn, wsp_v, wsn_v = lax.fori_loop(0, d // (_NB * u), collect,
                                             (zi, zi, zf, zf))
    c1p = jnp.max(offp)
    c1n = jnp.max(offn)
    wsp = jnp.sum(wsp_v)
    wsn = jnp.sum(wsn_v)

    tp, gt2p, neq_p, ws2p = _refine(cand_a, hist2, c1p, k1 - hi1p,
                                    b1p << 23, d)
    tn, gt2n, neq_n, ws2n = _refine(cand_b, hist2, c1n, k2 - hi1n,
                                    b1n << 23, d)
    need_p = k1 - hi1p - gt2p
    need_n = k2 - hi1n - gt2n
    tvp = lax.bitcast_convert_type(tp, jnp.float32)
    tvn = lax.bitcast_convert_type(tn, jnp.float32)
    wsum_p = wsp + ws2p + need_p.astype(jnp.float32) * tvp
    wsum_n = wsn + ws2n + need_n.astype(jnp.float32) * tvn

    ptmp = factor * (sum_p - wsum_p)
    ntmp = factor * (sum_n - wsum_n)

    def out_fast(_):
        ntvn = -tvn

        def body(i, c):
            bs = i * (_NB * u)
            for j in range(u):
                v = xv[pl.ds(bs + j * _NB, _NB)]
                winp = v >= tvp
                winn = v <= ntvn
                out = jnp.where(winp, v + ptmp, 0.0)
                out = jnp.where(winn, v - ntmp, out)
                xv[pl.ds(bs + j * _NB, _NB)] = out
            return c

        return lax.fori_loop(0, d // (_NB * u), body, 0)

    def out_slow(_):
        needpv = jnp.full((_NB,), need_p)
        neednv = jnp.full((_NB,), need_n)

        def body(i, carry):
            tkp, tkn = carry
            bs = i * (_NB * u)
            for j in range(u):
                v = xv[pl.ds(bs + j * _NB, _NB)]
                p = jnp.maximum(v, 0.0)
                n = jnp.maximum(0.0, -v)
                bp = lax.bitcast_convert_type(p, jnp.int32) & 0x7FFFFFFF
                bn = lax.bitcast_convert_type(n, jnp.int32) & 0x7FFFFFFF
                eqp = bp == tp
                eqn = bn == tn
                icp = plsc.cumsum(eqp.astype(jnp.int32))
                icn = plsc.cumsum(eqn.astype(jnp.int32))
                winp = (bp > tp) | (eqp & ((icp + tkp) <= needpv))
                winn = (bn > tn) | (eqn & ((icn + tkn) <= neednv))
                tkp = tkp + plsc.all_reduce_population_count(eqp)
                tkn = tkn + plsc.all_reduce_population_count(eqn)
                out = (jnp.where(winp, p + ptmp, 0.0)
                       - jnp.where(winn, n + ntmp, 0.0))
                xv[pl.ds(bs + j * _NB, _NB)] = out
            return tkp, tkn

        lax.fori_loop(0, d // (_NB * u), body, (zi, zi))
        return 0

    fast = ((neq_p == need_p) & (neq_n == need_n) & (tp > 0) & (tn > 0))
    lax.cond(fast, out_fast, out_slow, operand=None)


def _run_sc(x):
    b, d = x.shape
    topk = min(_TOPK, d)
    k1 = topk // 2
    k2 = topk - k1

    info = plsc.get_sparse_core_info()
    nw = info.num_cores * info.num_subcores
    rows_per = (b + nw - 1) // nw
    mesh = plsc.VectorSubcoreMesh(core_axis_name="c", subcore_axis_name="s")

    @functools.partial(
        pl.kernel,
        out_type=jax.ShapeDtypeStruct((b, d), jnp.float32),
        mesh=mesh,
        compiler_params=pltpu.CompilerParams(needs_layout_passes=False),
        scratch_types=[
            pltpu.VMEM((d,), jnp.float32),
            pltpu.VMEM((d + _NB,), jnp.int32),
            pltpu.VMEM((d + _NB,), jnp.int32),
            pltpu.VMEM((8192,), jnp.int32),
            pltpu.VMEM((8192,), jnp.int32),
        ],
    )
    def run(x_hbm, out_hbm, xv, cand_a, cand_b, hist1, hist2):
        wid = lax.axis_index("s") * info.num_cores + lax.axis_index("c")

        def row_loop(j, c):
            row = wid * rows_per + j

            @pl.when(row < b)
            def _():
                pltpu.sync_copy(x_hbm.at[row], xv)
                _process_row(xv, cand_a, cand_b, hist1, hist2, k1, k2,
                             _FACTOR, d)
                pltpu.sync_copy(xv, out_hbm.at[row])

            return c

        lax.fori_loop(0, rows_per, row_loop, 0)

    return run(x)


def _tc_block(x_ref, o_ref, *, k1, k2, factor, idx_bits):
    x = x_ref[...]
    r, d = x.shape
    pos = jnp.maximum(x, 0.0)
    neg = jnp.maximum(-x, 0.0)
    pos_bits = jax.lax.bitcast_convert_type(pos, jnp.int32) & 0x7FFFFFFF
    neg_bits = jax.lax.bitcast_convert_type(neg, jnp.int32) & 0x7FFFFFFF

    def count_ge(bits, thr):
        return jnp.sum((bits >= thr).astype(jnp.int32), axis=1, keepdims=True)

    def val_step(i, carry):
        tp, tn = carry
        bit = jnp.int32(1) << (30 - i)
        candp = tp | bit
        candn = tn | bit
        tp = jnp.where(count_ge(pos_bits, candp) >= k1, candp, tp)
        tn = jnp.where(count_ge(neg_bits, candn) >= k2, candn, tn)
        return tp, tn

    zeros = jnp.zeros((r, 1), jnp.int32)
    tp, tn = jax.lax.fori_loop(0, 31, val_step, (zeros, zeros))

    key = (d - 1) - jax.lax.broadcasted_iota(jnp.int32, (r, d), 1)
    eq_p = pos_bits == tp
    eq_n = neg_bits == tn

    def tie_search(_):
        cnt_gt_p = count_ge(pos_bits, tp + 1)
        cnt_gt_n = count_ge(neg_bits, tn + 1)
        need_p = k1 - cnt_gt_p
        need_n = k2 - cnt_gt_n

        def idx_step(i, carry):
            kp, kn = carry
            bit = jnp.int32(1) << (idx_bits - 1 - i)
            candp = kp | bit
            candn = kn | bit
            cp = jnp.sum((eq_p & (key >= candp)).astype(jnp.int32), axis=1,
                         keepdims=True)
            cn = jnp.sum((eq_n & (key >= candn)).astype(jnp.int32), axis=1,
                         keepdims=True)
            kp = jnp.where(cp >= need_p, candp, kp)
            kn = jnp.where(cn >= need_n, candn, kn)
            return kp, kn

        return jax.lax.fori_loop(0, idx_bits, idx_step, (zeros, zeros))

    no_ties = jnp.logical_and(jnp.all(count_ge(pos_bits, tp) == k1),
                              jnp.all(count_ge(neg_bits, tn) == k2))
    kp, kn = jax.lax.cond(no_ties, lambda _: (zeros, zeros), tie_search,
                          operand=None)

    maskp = (pos_bits > tp) | (eq_p & (key >= kp))
    maskn = (neg_bits > tn) | (eq_n & (key >= kn))

    loser_p = jnp.sum(jnp.where(maskp, 0.0, pos), axis=1, keepdims=True)
    loser_n = jnp.sum(jnp.where(maskn, 0.0, neg), axis=1, keepdims=True)
    ptmp = factor * loser_p
    ntmp = factor * loser_n

    o_ref[...] = (jnp.where(maskp, pos + ptmp, 0.0)
                  - jnp.where(maskn, neg + ntmp, 0.0))


def _block_rows(b):
    for r in (16, 8, 4, 2, 1):
        if b % r == 0:
            return r
    return 1


def _run_tc(x):
    b, d = x.shape
    topk = min(_TOPK, d)
    k1 = topk // 2
    k2 = topk - k1
    r = _block_rows(b)
    idx_bits = max(1, (d - 1).bit_length())
    body = functools.partial(_tc_block, k1=k1, k2=k2, factor=_FACTOR,
                             idx_bits=idx_bits)
    return pl.pallas_call(
        body,
        grid=(b // r,),
        in_specs=[pl.BlockSpec((r, d), lambda i: (i, 0))],
        out_specs=pl.BlockSpec((r, d), lambda i: (i, 0)),
        out_shape=jax.ShapeDtypeStruct((b, d), jnp.float32),
    )(x)


_TC_FRAC = 0.625


@jax.jit
def kernel(x):
    b, d = x.shape
    tc_rows = (int(b * _TC_FRAC) // 16) * 16
    if tc_rows <= 0:
        return _run_sc(x)
    if tc_rows >= b:
        return _run_tc(x)
    tc_out = _run_tc(x[:tc_rows])
    sc_out = _run_sc(x[tc_rows:])
    return jnp.concatenate([tc_out, sc_out], axis=0)


# speedup vs baseline: 1.0240x; 1.0240x over previous
import functools

import jax
import jax.numpy as jnp
from jax import lax
from jax.experimental import pallas as pl
from jax.experimental.pallas import tpu as pltpu
from jax.experimental.pallas import tpu_sc as plsc

_FACTOR = 6.26
_TOPK = 128
_NB = 16
_UNROLL = 8


def _zero_hist(hist_ref, base, nwords):
    z = jnp.zeros((_NB,), jnp.int32)

    def body(i, c):
        off = base + i * (_NB * 8)
        for j in range(8):
            hist_ref[pl.ds(off + j * _NB, _NB)] = z
        return c

    lax.fori_loop(0, nwords // (_NB * 8), body, 0)


def _find_bucket(hist_ref, base, r, nbuckets, ascending):
    nch = nbuckets // _NB

    def chunk_sum(c):
        off = base + c * (_NB * _NB)
        acc = hist_ref[pl.ds(off, _NB)]
        for j in range(1, _NB):
            acc = acc + hist_ref[pl.ds(off + j * _NB, _NB)]
        return jnp.sum(acc)

    def chunk_body(i, carry):
        cum, cstar, cnt_hi = carry
        c = i if ascending else (nch - 1) - i
        tot = chunk_sum(c)
        ncum = cum + tot
        hit = (cstar < 0) & (ncum >= r)
        cstar = jnp.where(hit, c, cstar)
        cnt_hi = jnp.where(hit, cum, cnt_hi)
        return ncum, cstar, cnt_hi

    cum_all, cstar, cnt_hi = lax.fori_loop(
        0, nch, chunk_body, (jnp.int32(0), jnp.int32(-1), jnp.int32(0)))

    safe_cstar = jnp.maximum(cstar, 0)

    def bucket_body(i, carry):
        cum, rstar, cnt_hi2, cnt_at = carry
        row = safe_cstar * _NB + (i if ascending else (_NB - 1) - i)
        tot = jnp.sum(hist_ref[pl.ds(base + row * _NB, _NB)])
        ncum = cum + tot
        hit = (rstar < 0) & (ncum >= r)
        rstar = jnp.where(hit, row, rstar)
        cnt_hi2 = jnp.where(hit, cum, cnt_hi2)
        cnt_at = jnp.where(hit, tot, cnt_at)
        return ncum, rstar, cnt_hi2, cnt_at

    _, rstar, cnt_hi2, cnt_at = lax.fori_loop(
        0, _NB, bucket_body,
        (cnt_hi, jnp.int32(-1), jnp.int32(0), jnp.int32(0)))
    rstar = jnp.where(cstar < 0, jnp.int32(-1), rstar)

    row0 = (nbuckets - 1) if ascending else 0
    tot0 = jnp.sum(hist_ref[pl.ds(base + row0 * _NB, _NB)])
    found = rstar >= 0
    bucket = jnp.where(found,
                       ((nbuckets - 1) - rstar) if ascending else rstar,
                       jnp.int32(0))
    cnt_hi_out = jnp.where(found, cnt_hi2, cum_all - tot0)
    cnt_at = jnp.where(found, cnt_at, tot0)
    return bucket, cnt_hi_out, cnt_at


_LEVELS = ((15, 256, 7, 256, False), (7, 256, 0, 128, False),
           (0, 128, 0, 0, True))


def _refine(cand, hist, c1, r, tbits, d):
    lane = lax.iota(jnp.int32, _NB)
    ones = jnp.full((_NB,), 1, jnp.int32)
    zi = jnp.zeros((_NB,), jnp.int32)
    zf = jnp.zeros((_NB,), jnp.float32)

    _zero_hist(hist, 0, 4096)
    nfull = c1 // _NB
    rem = c1 - nfull * _NB

    def h2_body(i, c):
        bits = cand[pl.ds(i * _NB, _NB)]
        a = (((bits >> 15) & 255) << 4) | lane
        plsc.addupdate_scatter(hist, [a], ones, mask=None)
        return c

    lax.fori_loop(0, nfull, h2_body, 0)
    bits = cand[pl.ds(nfull * _NB, _NB)]
    a = (((bits >> 15) & 255) << 4) | lane
    plsc.addupdate_scatter(hist, [a], ones, mask=lane < rem)

    cnt_gt = jnp.int32(0)
    n_eq = jnp.int32(0)
    ws = jnp.float32(0.0)
    cnum = c1
    hbase, hnext = 0, 4096
    for shift, width, nshift, nwidth, last in _LEVELS:
        bl, cnt_hi, cnt_at = _find_bucket(hist, hbase, r, width, False)
        tbits = tbits | (bl << shift)
        cnt_gt = cnt_gt + cnt_hi
        r = r - cnt_hi
        if last:
            n_eq = cnt_at
        if not last:
            _zero_hist(hist, hnext, 4096)
        nfull = cnum // _NB
        rem = cnum - nfull * _NB

        def sel_chunk(bits, valid, carry, shift=shift, width=width,
                      nshift=nshift, nwidth=nwidth, bl=bl, last=last,
                      hnext=hnext):
            offv, wsv = carry
            bk = (bits >> shift) & (width - 1)
            wsv = wsv + jnp.where(valid & (bk > bl),
                                  lax.bitcast_convert_type(bits, jnp.float32),
                                  0.0)
            if not last:
                m = valid & (bk == bl)
                mi = m.astype(jnp.int32)
                tgt = offv + (plsc.cumsum(mi) - mi)
                plsc.store_scatter(cand, [tgt], bits, mask=m)
                a2 = ((((bits >> nshift) & (nwidth - 1)) << 4) | lane) + hnext
                plsc.addupdate_scatter(hist, [a2], ones, mask=m)
                offv = offv + plsc.all_reduce_population_count(m)
            return offv, wsv

        def sel_body(i, carry, sel_chunk=sel_chunk):
            bits = cand[pl.ds(i * _NB, _NB)]
            return sel_chunk(bits, jnp.full((_NB,), True), carry)

        offv, wsv = lax.fori_loop(0, nfull, sel_body, (zi, zf))
        tail = cand[pl.ds(nfull * _NB, _NB)]
        offv, wsv = sel_chunk(tail, lane < rem, (offv, wsv))
        ws = ws + jnp.sum(wsv)
        if not last:
            cnum = jnp.max(offv)
        hbase, hnext = hnext, hbase

    return tbits, cnt_gt, n_eq, ws


def _process_row(xv, cand_a, cand_b, hist1, hist2, k1, k2, factor, d):
    lane = lax.iota(jnp.int32, _NB)
    ones = jnp.full((_NB,), 1, jnp.int32)
    _zero_hist(hist1, 0, 8192)
    u = _UNROLL if d % (_NB * _UNROLL) == 0 else 1

    def pass_a(i, carry):
        sv, sa = carry
        bs = i * (_NB * u)
        for j in range(u):
            v = xv[pl.ds(bs + j * _NB, _NB)]
            braw = lax.bitcast_convert_type(v, jnp.int32)
            babs = braw & 0x7FFFFFFF
            av = lax.bitcast_convert_type(babs, jnp.float32)
            sv = sv + v
            sa = sa + av
            bucket = babs >> 23
            row = jnp.where(v >= 0.0, 256 + bucket, 255 - bucket)
            addr = (row << 4) | lane
            plsc.addupdate_scatter(hist1, [addr], ones, mask=None)
        return sv, sa

    zf = jnp.zeros((_NB,), jnp.float32)
    zi = jnp.zeros((_NB,), jnp.int32)
    sv, sa = lax.fori_loop(0, d // (_NB * u), pass_a, (zf, zf))
    tsv = jnp.sum(sv)
    tsa = jnp.sum(sa)
    sum_p = 0.5 * (tsa + tsv)
    sum_n = 0.5 * (tsa - tsv)

    b1p, hi1p, _ = _find_bucket(hist1, 4096, k1, 256, False)
    b1n, hi1n, _ = _find_bucket(hist1, 0, k2, 256, True)

    def collect(i, carry):
        offp, offn, wsp, wsn = carry
        bs = i * (_NB * u)
        for j in range(u):
            v = xv[pl.ds(bs + j * _NB, _NB)]
            braw = lax.bitcast_convert_type(v, jnp.int32)
            babs = braw & 0x7FFFFFFF
            pos = v > 0.0
            neg = v < 0.0
            bkp = jnp.where(pos, babs, 0)
            bkn = jnp.where(neg, babs, 0)
            bucketp = bkp >> 23
            bucketn = bkn >> 23
            mp = bucketp == b1p
            mn = bucketn == b1n
            mpi = mp.astype(jnp.int32)
            mni = mn.astype(jnp.int32)
            tgtp = offp + (plsc.cumsum(mpi) - mpi)
            tgtn = offn + (plsc.cumsum(mni) - mni)
            plsc.store_scatter(cand_a, [tgtp], bkp, mask=mp)
            plsc.store_scatter(cand_b, [tgtn], bkn, mask=mn)
            offp = offp + plsc.all_reduce_population_count(mp)
            offn = offn + plsc.all_reduce_population_count(mn)
            pv = lax.bitcast_convert_type(bkp, jnp.float32)
            nv = lax.bitcast_convert_type(bkn, jnp.float32)
            wsp = wsp + jnp.where(bucketp > b1p, pv, 0.0)
            wsn = wsn + jnp.where(bucketn > b1n, nv, 0.0)
        return offp, offn, wsp, wsn

    offp, offn, wsp_v, wsn_v = lax.fori_loop(0, d // (_NB * u), collect,
                                             (zi, zi, zf, zf))
    c1p = jnp.max(offp)
    c1n = jnp.max(offn)
    wsp = jnp.sum(wsp_v)
    wsn = jnp.sum(wsn_v)

    tp, gt2p, neq_p, ws2p = _refine(cand_a, hist2, c1p, k1 - hi1p,
                                    b1p << 23, d)
    tn, gt2n, neq_n, ws2n = _refine(cand_b, hist2, c1n, k2 - hi1n,
                                    b1n << 23, d)
    need_p = k1 - hi1p - gt2p
    need_n = k2 - hi1n - gt2n
    tvp = lax.bitcast_convert_type(tp, jnp.float32)
    tvn = lax.bitcast_convert_type(tn, jnp.float32)
    wsum_p = wsp + ws2p + need_p.astype(jnp.float32) * tvp
    wsum_n = wsn + ws2n + need_n.astype(jnp.float32) * tvn

    ptmp = factor * (sum_p - wsum_p)
    ntmp = factor * (sum_n - wsum_n)

    def out_fast(_):
        ntvn = -tvn

        def body(i, c):
            bs = i * (_NB * u)
            for j in range(u):
                v = xv[pl.ds(bs + j * _NB, _NB)]
                winp = v >= tvp
                winn = v <= ntvn
                out = jnp.where(winp, v + ptmp, 0.0)
                out = jnp.where(winn, v - ntmp, out)
                xv[pl.ds(bs + j * _NB, _NB)] = out
            return c

        return lax.fori_loop(0, d // (_NB * u), body, 0)

    def out_slow(_):
        needpv = jnp.full((_NB,), need_p)
        neednv = jnp.full((_NB,), need_n)

        def body(i, carry):
            tkp, tkn = carry
            bs = i * (_NB * u)
            for j in range(u):
                v = xv[pl.ds(bs + j * _NB, _NB)]
                p = jnp.maximum(v, 0.0)
                n = jnp.maximum(0.0, -v)
                bp = lax.bitcast_convert_type(p, jnp.int32) & 0x7FFFFFFF
                bn = lax.bitcast_convert_type(n, jnp.int32) & 0x7FFFFFFF
                eqp = bp == tp
                eqn = bn == tn
                icp = plsc.cumsum(eqp.astype(jnp.int32))
                icn = plsc.cumsum(eqn.astype(jnp.int32))
                winp = (bp > tp) | (eqp & ((icp + tkp) <= needpv))
                winn = (bn > tn) | (eqn & ((icn + tkn) <= neednv))
                tkp = tkp + plsc.all_reduce_population_count(eqp)
                tkn = tkn + plsc.all_reduce_population_count(eqn)
                out = (jnp.where(winp, p + ptmp, 0.0)
                       - jnp.where(winn, n + ntmp, 0.0))
                xv[pl.ds(bs + j * _NB, _NB)] = out
            return tkp, tkn

        lax.fori_loop(0, d // (_NB * u), body, (zi, zi))
        return 0

    fast = ((neq_p == need_p) & (neq_n == need_n) & (tp > 0) & (tn > 0))
    lax.cond(fast, out_fast, out_slow, operand=None)


def _run_sc(x):
    b, d = x.shape
    topk = min(_TOPK, d)
    k1 = topk // 2
    k2 = topk - k1

    info = plsc.get_sparse_core_info()
    nw = info.num_cores * info.num_subcores
    rows_per = (b + nw - 1) // nw
    mesh = plsc.VectorSubcoreMesh(core_axis_name="c", subcore_axis_name="s")

    @functools.partial(
        pl.kernel,
        out_type=jax.ShapeDtypeStruct((b, d), jnp.float32),
        mesh=mesh,
        compiler_params=pltpu.CompilerParams(needs_layout_passes=False),
        scratch_types=[
            pltpu.VMEM((d,), jnp.float32),
            pltpu.VMEM((d + _NB,), jnp.int32),
            pltpu.VMEM((d + _NB,), jnp.int32),
            pltpu.VMEM((8192,), jnp.int32),
            pltpu.VMEM((8192,), jnp.int32),
        ],
    )
    def run(x_hbm, out_hbm, xv, cand_a, cand_b, hist1, hist2):
        wid = lax.axis_index("s") * info.num_cores + lax.axis_index("c")

        def row_loop(j, c):
            row = wid * rows_per + j

            @pl.when(row < b)
            def _():
                pltpu.sync_copy(x_hbm.at[row], xv)
                _process_row(xv, cand_a, cand_b, hist1, hist2, k1, k2,
                             _FACTOR, d)
                pltpu.sync_copy(xv, out_hbm.at[row])

            return c

        lax.fori_loop(0, rows_per, row_loop, 0)

    return run(x)


def _tc_block(x_ref, o_ref, *, k1, k2, factor, idx_bits):
    x = x_ref[...]
    r, d = x.shape
    pos = jnp.maximum(x, 0.0)
    neg = jnp.maximum(-x, 0.0)
    pos_bits = jax.lax.bitcast_convert_type(pos, jnp.int32) & 0x7FFFFFFF
    neg_bits = jax.lax.bitcast_convert_type(neg, jnp.int32) & 0x7FFFFFFF

    def count_ge(bits, thr):
        return jnp.sum((bits >= thr).astype(jnp.int32), axis=1, keepdims=True)

    def val_step(i, carry):
        tp, tn = carry
        bit = jnp.int32(1) << (30 - i)
        candp = tp | bit
        candn = tn | bit
        tp = jnp.where(count_ge(pos_bits, candp) >= k1, candp, tp)
        tn = jnp.where(count_ge(neg_bits, candn) >= k2, candn, tn)
        return tp, tn

    zeros = jnp.zeros((r, 1), jnp.int32)
    tp, tn = jax.lax.fori_loop(0, 31, val_step, (zeros, zeros))

    key = (d - 1) - jax.lax.broadcasted_iota(jnp.int32, (r, d), 1)
    eq_p = pos_bits == tp
    eq_n = neg_bits == tn

    def tie_search(_):
        cnt_gt_p = count_ge(pos_bits, tp + 1)
        cnt_gt_n = count_ge(neg_bits, tn + 1)
        need_p = k1 - cnt_gt_p
        need_n = k2 - cnt_gt_n

        def idx_step(i, carry):
            kp, kn = carry
            bit = jnp.int32(1) << (idx_bits - 1 - i)
            candp = kp | bit
            candn = kn | bit
            cp = jnp.sum((eq_p & (key >= candp)).astype(jnp.int32), axis=1,
                         keepdims=True)
            cn = jnp.sum((eq_n & (key >= candn)).astype(jnp.int32), axis=1,
                         keepdims=True)
            kp = jnp.where(cp >= need_p, candp, kp)
            kn = jnp.where(cn >= need_n, candn, kn)
            return kp, kn

        return jax.lax.fori_loop(0, idx_bits, idx_step, (zeros, zeros))

    no_ties = jnp.logical_and(jnp.all(count_ge(pos_bits, tp) == k1),
                              jnp.all(count_ge(neg_bits, tn) == k2))
    kp, kn = jax.lax.cond(no_ties, lambda _: (zeros, zeros), tie_search,
                          operand=None)

    maskp = (pos_bits > tp) | (eq_p & (key >= kp))
    maskn = (neg_bits > tn) | (eq_n & (key >= kn))

    loser_p = jnp.sum(jnp.where(maskp, 0.0, pos), axis=1, keepdims=True)
    loser_n = jnp.sum(jnp.where(maskn, 0.0, neg), axis=1, keepdims=True)
    ptmp = factor * loser_p
    ntmp = factor * loser_n

    o_ref[...] = (jnp.where(maskp, pos + ptmp, 0.0)
                  - jnp.where(maskn, neg + ntmp, 0.0))


def _block_rows(b):
    for r in (16, 8, 4, 2, 1):
        if b % r == 0:
            return r
    return 1


def _run_tc(x):
    b, d = x.shape
    topk = min(_TOPK, d)
    k1 = topk // 2
    k2 = topk - k1
    r = _block_rows(b)
    idx_bits = max(1, (d - 1).bit_length())
    body = functools.partial(_tc_block, k1=k1, k2=k2, factor=_FACTOR,
                             idx_bits=idx_bits)
    return pl.pallas_call(
        body,
        grid=(b // r,),
        in_specs=[pl.BlockSpec((r, d), lambda i: (i, 0))],
        out_specs=pl.BlockSpec((r, d), lambda i: (i, 0)),
        out_shape=jax.ShapeDtypeStruct((b, d), jnp.float32),
    )(x)


_TC_FRAC = 0.5


@jax.jit
def kernel(x):
    b, d = x.shape
    tc_rows = (int(b * _TC_FRAC) // 16) * 16
    if tc_rows <= 0:
        return _run_sc(x)
    if tc_rows >= b:
        return _run_tc(x)
    tc_out = _run_tc(x[:tc_rows])
    sc_out = _run_sc(x[tc_rows:])
    return jnp.concatenate([tc_out, sc_out], axis=0)
